# baseline (speedup 1.0000x reference)
import functools

import jax
import jax.numpy as jnp
from jax import lax
from jax.experimental import pallas as pl
from jax.experimental.pallas import tpu as pltpu
from jax.experimental.pallas import tpu_sc as plsc

_L = 16


def _make_sc_kernel(V, D, B):
    NW = 32
    bpw = B // NW
    n_rows = 2 * bpw
    SLOTS = 2 * _L
    n_chunks = n_rows // SLOTS
    DC = D // _L
    DH = D // 8

    mesh = plsc.VectorSubcoreMesh(core_axis_name="c", subcore_axis_name="s")

    @functools.partial(
        pl.kernel,
        out_type=jax.ShapeDtypeStruct((B,), jnp.float32),
        mesh=mesh,
        scratch_types=[
            pltpu.VMEM((n_rows,), jnp.int32),
            pltpu.VMEM((DH, 8, SLOTS * _L), jnp.float32),
            pltpu.VMEM((D,), jnp.float32),
            pltpu.VMEM((_L, _L), jnp.float32),
            pltpu.VMEM((bpw,), jnp.float32),
            pltpu.SemaphoreType.DMA,
        ],
        compiler_params=pltpu.CompilerParams(needs_layout_passes=False),
    )
    def run(embt_hbm, idx_hbm, r_hbm, out_hbm, idx_v, gbuf, r_v, p_v, out_v, sem):
        wid = lax.axis_index("s") * 2 + lax.axis_index("c")
        base = wid * n_rows

        pltpu.sync_copy(idx_hbm.at[pl.ds(base, n_rows)], idx_v)
        pltpu.sync_copy(r_hbm, r_v)

        r_regs = [r_v[pl.ds(c * _L, _L)] for c in range(DC)]
        iota = lax.iota(jnp.int32, _L)
        kl_idx = [
            (
                lax.shift_right_logical(c * _L + iota, 3),
                lax.bitwise_and(c * _L + iota, 7),
            )
            for c in range(DC)
        ]

        def chunk_body(chunk, carry):
            vvecs = [
                idx_v[pl.ds(chunk * SLOTS + h * _L, _L)] for h in range(SLOTS // _L)
            ]
            vls = [lax.bitwise_and(vv, jnp.int32(15)) for vv in vvecs]

            def fire_h(h, carry2):
                vvec = idx_v[pl.ds(chunk * SLOTS + h * _L, _L)]
                vt_vec = lax.shift_right_logical(vvec, 7) * 128
                w16_vec = lax.bitwise_and(
                    lax.shift_right_logical(vvec, 4), jnp.int32(7)
                )
                for j in range(_L):
                    vt = pl.multiple_of(vt_vec[j], 128)
                    w16 = w16_vec[j]
                    dst_off = pl.multiple_of((h * _L + j) * _L, _L)
                    dst = gbuf.at[:, :, pl.ds(dst_off, _L)]
                    for w in range(8):

                        @pl.when(w16 == w)
                        def _fire(vt=vt, w=w, dst=dst):
                            pltpu.async_copy(
                                embt_hbm.at[:, :, pl.ds(vt, 128)].at[
                                    :, :, pl.ds(w * _L, _L)
                                ],
                                dst,
                                sem,
                            )

                return carry2

            lax.fori_loop(0, SLOTS // _L, fire_h, 0, unroll=False)

            def drain_body(n, carry2):
                pltpu.make_async_copy(
                    embt_hbm.at[:, :, pl.ds(0, _L)],
                    gbuf.at[:, :, pl.ds(0, _L)],
                    sem,
                ).wait()
                return carry2

            lax.fori_loop(0, SLOTS, drain_body, 0, unroll=False)

            for j in range(_L):
                s_lane = jnp.broadcast_to(vls[(2 * j) // _L][(2 * j) % _L], (_L,))
                o_lane = jnp.broadcast_to(vls[(2 * j + 1) // _L][(2 * j + 1) % _L], (_L,))
                s_slot = jnp.broadcast_to(jnp.int32(2 * j), (_L,))
                o_slot = jnp.broadcast_to(jnp.int32(2 * j + 1), (_L,))
                acc = None
                for c in range(DC):
                    k_idx, l_idx = kl_idx[c]
                    s_c = plsc.load_gather(gbuf, [k_idx, l_idx, s_slot * _L + s_lane])
                    o_c = plsc.load_gather(gbuf, [k_idx, l_idx, o_slot * _L + o_lane])
                    t = (s_c * o_c) * r_regs[c]
                    acc = t if acc is None else acc + t
                p_v[j, :] = acc

            accv = jnp.zeros((_L,), jnp.float32)
            for l in range(_L):
                col = plsc.load_gather(p_v, [iota, jnp.full((_L,), l, jnp.int32)])
                accv = accv + col
            sig = 1.0 / (1.0 + jnp.exp(-accv))
            out_v[pl.ds(chunk * _L, _L)] = sig
            return carry

        lax.fori_loop(0, n_chunks, chunk_body, 0, unroll=False)

        pltpu.sync_copy(out_v, out_hbm.at[pl.ds(wid * bpw, bpw)])

    return run


def kernel(emb, batch_ind, r):
    V, D = emb.shape
    B = batch_ind.shape[0]
    embt3 = emb.T.reshape(D // 8, 8, V)
    idx_flat = batch_ind.reshape(2 * B)
    run = _make_sc_kernel(V, D, B)
    return run(embt3, idx_flat, r)


# speedup vs baseline: 1.0075x; 1.0075x over previous
import functools

import jax
import jax.numpy as jnp
from jax import lax
from jax.experimental import pallas as pl
from jax.experimental.pallas import tpu as pltpu
from jax.experimental.pallas import tpu_sc as plsc

_L = 16


def _make_sc_kernel(V, D, B):
    NW = 32
    bpw = B // NW
    n_rows = 2 * bpw
    SLOTS = 2 * _L
    n_chunks = n_rows // SLOTS
    DC = D // _L
    DH = D // 8

    mesh = plsc.VectorSubcoreMesh(core_axis_name="c", subcore_axis_name="s")

    @functools.partial(
        pl.kernel,
        out_type=jax.ShapeDtypeStruct((B,), jnp.float32),
        mesh=mesh,
        scratch_types=[
            pltpu.VMEM((n_rows,), jnp.int32),
            pltpu.VMEM((DH, 8, SLOTS * _L), jnp.float32),
            pltpu.VMEM((D,), jnp.float32),
            pltpu.VMEM((_L, _L), jnp.float32),
            pltpu.VMEM((bpw,), jnp.float32),
            pltpu.SemaphoreType.DMA,
            pltpu.SemaphoreType.DMA,
            pltpu.SemaphoreType.DMA,
            pltpu.SemaphoreType.DMA,
        ],
        compiler_params=pltpu.CompilerParams(needs_layout_passes=False),
    )
    def run(
        embt_hbm, idx_hbm, r_hbm, out_hbm,
        idx_v, gbuf, r_v, p_v, out_v, sem, sem1, sem2, sem3,
    ):
        sems = (sem, sem1, sem2, sem3)
        wid = lax.axis_index("s") * 2 + lax.axis_index("c")
        base = wid * n_rows

        pltpu.sync_copy(idx_hbm.at[pl.ds(base, n_rows)], idx_v)
        pltpu.sync_copy(r_hbm, r_v)

        r_regs = [r_v[pl.ds(c * _L, _L)] for c in range(DC)]
        iota = lax.iota(jnp.int32, _L)
        kl_idx = [
            (
                lax.shift_right_logical(c * _L + iota, 3),
                lax.bitwise_and(c * _L + iota, 7),
            )
            for c in range(DC)
        ]

        def chunk_body(chunk, carry):
            vvecs = [
                idx_v[pl.ds(chunk * SLOTS + h * _L, _L)] for h in range(SLOTS // _L)
            ]
            vls = [lax.bitwise_and(vv, jnp.int32(15)) for vv in vvecs]

            def fire_h(h, carry2):
                vvec = idx_v[pl.ds(chunk * SLOTS + h * _L, _L)]
                vt_vec = lax.shift_right_logical(vvec, 7) * 128
                w16_vec = lax.bitwise_and(
                    lax.shift_right_logical(vvec, 4), jnp.int32(7)
                )
                for j in range(_L):
                    vt = pl.multiple_of(vt_vec[j], 128)
                    w16 = w16_vec[j]
                    dst_off = pl.multiple_of((h * _L + j) * _L, _L)
                    dst = gbuf.at[:, :, pl.ds(dst_off, _L)]
                    for w in range(8):

                        @pl.when(w16 == w)
                        def _fire(vt=vt, w=w, dst=dst, s=sems[j % 4]):
                            pltpu.async_copy(
                                embt_hbm.at[:, :, pl.ds(vt, 128)].at[
                                    :, :, pl.ds(w * _L, _L)
                                ],
                                dst,
                                s,
                            )

                return carry2

            lax.fori_loop(0, SLOTS // _L, fire_h, 0, unroll=False)

            def drain_body(n, carry2):
                for s in sems:
                    pltpu.make_async_copy(
                        embt_hbm.at[:, :, pl.ds(0, _L)],
                        gbuf.at[:, :, pl.ds(0, _L)],
                        s,
                    ).wait()
                return carry2

            lax.fori_loop(0, SLOTS // 4, drain_body, 0, unroll=False)

            for j in range(_L):
                s_lane = jnp.broadcast_to(vls[(2 * j) // _L][(2 * j) % _L], (_L,))
                o_lane = jnp.broadcast_to(vls[(2 * j + 1) // _L][(2 * j + 1) % _L], (_L,))
                s_slot = jnp.broadcast_to(jnp.int32(2 * j), (_L,))
                o_slot = jnp.broadcast_to(jnp.int32(2 * j + 1), (_L,))
                acc = None
                for c in range(DC):
                    k_idx, l_idx = kl_idx[c]
                    s_c = plsc.load_gather(gbuf, [k_idx, l_idx, s_slot * _L + s_lane])
                    o_c = plsc.load_gather(gbuf, [k_idx, l_idx, o_slot * _L + o_lane])
                    t = (s_c * o_c) * r_regs[c]
                    acc = t if acc is None else acc + t
                p_v[j, :] = acc

            accv = jnp.zeros((_L,), jnp.float32)
            for l in range(_L):
                col = plsc.load_gather(p_v, [iota, jnp.full((_L,), l, jnp.int32)])
                accv = accv + col
            sig = 1.0 / (1.0 + jnp.exp(-accv))
            out_v[pl.ds(chunk * _L, _L)] = sig
            return carry

        lax.fori_loop(0, n_chunks, chunk_body, 0, unroll=False)

        pltpu.sync_copy(out_v, out_hbm.at[pl.ds(wid * bpw, bpw)])

    return run


def kernel(emb, batch_ind, r):
    V, D = emb.shape
    B = batch_ind.shape[0]
    embt3 = emb.T.reshape(D // 8, 8, V)
    idx_flat = batch_ind.reshape(2 * B)
    run = _make_sc_kernel(V, D, B)
    return run(embt3, idx_flat, r)
